# baseline (speedup 1.0000x reference)
import functools

import jax
import jax.numpy as jnp
from jax import lax
from jax.experimental import pallas as pl
from jax.experimental.pallas import tpu as pltpu
from jax.experimental.pallas import tpu_sc as plsc

N = 10000
D = 128
E = 320000
NC, NS = 2, 16
NW = NC * NS
EPW = E // NW
NPAD = 10240
NROWB = NPAD // 128
EPW_PAD = NPAD

CHUNK = 128
NBUF = 4
EPT = E // NS
NVEC = EPT // 16
CAP = 20480
HALF = NPAD // NC
NPASS = 2
QROWS = HALF // NPASS
ACC_ROWS = QROWS + CHUNK
RRT = QROWS // NS

_MESH = plsc.VectorSubcoreMesh(core_axis_name="c", subcore_axis_name="s")
_SC_PARAMS = pltpu.CompilerParams(
    needs_layout_passes=False, use_tc_tiling_on_sc=False
)

_GROUPS = NROWB // 8
_GBINS = NPAD // _GROUPS


def _deg_body(src_hbm, dst_hbm, out_hbm, slots_hbm, idx_v, hist, rbuf, res):
    c = lax.axis_index("c")
    s = lax.axis_index("s")
    wid = s * NC + c

    zero16 = jnp.zeros((16,), jnp.float32)
    ones = jnp.ones((16,), jnp.float32)

    @pl.loop(0, NPAD // 16)
    def _z1(i):
        hist[pl.ds(i * 16, 16)] = zero16

    pltpu.sync_copy(src_hbm.at[wid], idx_v)

    @pl.loop(0, EPW_PAD // 16)
    def _hist_src(i):
        idx = idx_v[pl.ds(i * 16, 16)]
        plsc.addupdate_scatter(hist, [idx], ones)

    pltpu.sync_copy(hist, slots_hbm.at[c, 0, s])

    @pl.loop(0, NPAD // 16)
    def _z2(i):
        hist[pl.ds(i * 16, 16)] = zero16

    pltpu.sync_copy(dst_hbm.at[wid], idx_v)

    @pl.loop(0, EPW_PAD // 16)
    def _hist_dst(i):
        idx = idx_v[pl.ds(i * 16, 16)]
        plsc.addupdate_scatter(hist, [idx], ones)

    pltpu.sync_copy(hist, slots_hbm.at[c, 1, s])

    plsc.subcore_barrier()

    @pl.when(s < _GROUPS)
    def _reduce():
        for oidx in (0, 1):
            for r in range(NS):
                pltpu.sync_copy(
                    slots_hbm.at[c, oidx, r, pl.ds(s * _GBINS, _GBINS)],
                    rbuf.at[r])

            @pl.loop(0, _GBINS // 16)
            def _sum(k):
                acc = rbuf[0, pl.ds(k * 16, 16)]
                for r in range(1, NS):
                    acc = acc + rbuf[r, pl.ds(k * 16, 16)]
                res[k // 8, pl.ds((k % 8) * 16, 16)] = acc

            pltpu.sync_copy(res, out_hbm.at[c, oidx, pl.ds(s * 8, 8)])


_deg_kernel = pl.kernel(
    _deg_body,
    out_type=(
        jax.ShapeDtypeStruct((NC, 2, NROWB, 128), jnp.float32),
        jax.ShapeDtypeStruct((NC, 2, NS, NPAD), jnp.float32),
    ),
    mesh=_MESH,
    compiler_params=_SC_PARAMS,
    scratch_types=[
        pltpu.VMEM((EPW_PAD,), jnp.int32),
        pltpu.VMEM((NPAD,), jnp.float32),
        pltpu.VMEM((NS, _GBINS), jnp.float32),
        pltpu.VMEM((8, 128), jnp.float32),
    ],
)


def _agg_body(feat_hbm, src_hbm, dst_hbm, ndst_hbm, out_hbm,
              sbuf, dbuf, didx2, nbuf,
              rows0, rows1, rows2, rows3, acc, *sems):
    c = lax.axis_index("c")
    s = lax.axis_index("s")
    rows = (rows0, rows1, rows2, rows3)
    gsem = sems[:NBUF]
    ssem = sems[NBUF:]

    zero16 = jnp.zeros((16,), jnp.float32)
    iota = lax.iota(jnp.int32, 16)

    def stage_didx(k, b):
        for x in range(CHUNK // 16):
            didx2[b, pl.ds(x * 16, 16)] = dbuf[pl.ds(k * CHUNK + x * 16, 16)]

    def gather(k, b):
        return pltpu.async_copy(
            feat_hbm.at[c].at[sbuf.at[pl.ds(k * CHUNK, CHUNK)]],
            rows[b], gsem[b])

    def wait_gather(b):
        pltpu.make_async_copy(
            feat_hbm.at[c].at[sbuf.at[pl.ds(0, CHUNK)]],
            rows[b], gsem[b]).wait()

    def scatter(b):
        return pltpu.async_copy(
            rows[b], acc.at[didx2.at[b]], ssem[b], add=True)

    def wait_scatter(b):
        pltpu.make_async_copy(
            rows[b], acc.at[didx2.at[b]], ssem[b]).wait()

    for p in range(NPASS):
        for b in (0, 1):
            @pl.loop(0, CHUNK)
            def _zb(r):
                for x in range(D // 16):
                    rows[b][r, pl.ds(x * 16, 16)] = zero16

        zrows = ACC_ROWS // NS
        pltpu.sync_copy(rows0, acc.at[pl.ds(s * zrows, CHUNK)])
        pltpu.sync_copy(rows1.at[pl.ds(0, zrows - CHUNK)],
                        acc.at[pl.ds(s * zrows + CHUNK, zrows - CHUNK)])

        pltpu.sync_copy(src_hbm.at[s], sbuf.at[pl.ds(0, EPT)])
        pltpu.sync_copy(dst_hbm.at[s], dbuf.at[pl.ds(0, EPT)])
        base = c * HALF + p * QROWS
        pltpu.sync_copy(ndst_hbm.at[pl.ds(base + s * RRT, RRT)],
                        nbuf.at[pl.ds(0, RRT)])

        pad_s = iota + s * 16

        @pl.loop(0, NVEC)
        def _remap(i):
            dv = dbuf[pl.ds(i * 16, 16)] - base
            m = jnp.logical_and(dv >= 0, dv < QROWS)
            scrap = QROWS + ((iota + i * 16) & (CHUNK - 1))
            dbuf[pl.ds(i * 16, 16)] = jnp.where(m, dv, scrap)

        for t in range(NVEC, CAP // 16):
            sbuf[pl.ds(t * 16, 16)] = pad_s
            dbuf[pl.ds(t * 16, 16)] = QROWS + ((pad_s + t * 16) & (CHUNK - 1))

        nch = CAP // CHUNK

        plsc.subcore_barrier()

        gather(0, 0)

        @pl.loop(0, nch, step=NBUF)
        def _edges(m):
            for b in range(NBUF):
                k = m + b
                bn = (b + 1) % NBUF
                wait_gather(b)
                stage_didx(k, b)
                scatter(b)

                @pl.when(k + 1 - NBUF >= 0)
                def _():
                    wait_scatter(bn)

                @pl.when(k + 1 < nch)
                def _():
                    gather(k + 1, bn)

        for b in range(1, NBUF):
            wait_scatter(b)

        plsc.subcore_barrier()

        for off_b, nb in ((0, CHUNK), (CHUNK, RRT - CHUNK)):
            pltpu.sync_copy(acc.at[pl.ds(s * RRT + off_b, nb)],
                            rows0.at[pl.ds(0, nb)])

            @pl.loop(0, nb)
            def _scale(r):
                nv = nbuf[pl.ds(off_b + r, 16)]
                sc = nv[0]
                for x in range(D // 16):
                    rows0[r, pl.ds(x * 16, 16)] = (
                        rows0[r, pl.ds(x * 16, 16)] * sc)

            pltpu.sync_copy(
                rows0.at[pl.ds(0, nb)],
                out_hbm.at[c, pl.ds(base + s * RRT + off_b, nb)])


_agg_kernel = pl.kernel(
    _agg_body,
    out_type=jax.ShapeDtypeStruct((NC, NPAD, D), jnp.float32),
    mesh=_MESH,
    compiler_params=_SC_PARAMS,
    scratch_types=[
        pltpu.VMEM((CAP,), jnp.int32),
        pltpu.VMEM((CAP,), jnp.int32),
        pltpu.VMEM((NBUF, CHUNK), jnp.int32),
        pltpu.VMEM((RRT + 16,), jnp.float32),
        pltpu.VMEM((CHUNK, D), jnp.float32),
        pltpu.VMEM((CHUNK, D), jnp.float32),
        pltpu.VMEM((CHUNK, D), jnp.float32),
        pltpu.VMEM((CHUNK, D), jnp.float32),
        pltpu.VMEM_SHARED((ACC_ROWS, D), jnp.float32),
    ]
    + [pltpu.SemaphoreType.DMA] * (2 * NBUF),
)


def _scale_body(hs0, hs1, hd0, hd1, h_ref, feat_ref, ndst_ref):
    out_deg = hs0[...] + hs1[...]
    norm_src = 1.0 / jnp.maximum(out_deg, 1.0)
    feat_ref[...] = h_ref[...] * norm_src
    in_deg = hd0[...] + hd1[...]
    ndst_ref[...] = 1.0 / jnp.maximum(in_deg, 1.0)


def kernel(h, edge_index):
    ei = edge_index.astype(jnp.int32)
    src = ei[0]
    dst = ei[1]
    pad_idx = N + (jnp.arange(EPW_PAD - EPW, dtype=jnp.int32) % (NPAD - N))
    pad_blk = jnp.broadcast_to(pad_idx, (NW, EPW_PAD - EPW))
    src_p = jnp.concatenate([src.reshape(NW, EPW), pad_blk], axis=1)
    dst_p = jnp.concatenate([dst.reshape(NW, EPW), pad_blk], axis=1)
    src2 = src.reshape(NS, EPT)
    dst2 = dst.reshape(NS, EPT)
    h_pad = jnp.pad(h, ((0, NPAD - N), (0, 0)))

    hist, _ = _deg_kernel(src_p, dst_p)
    histc = hist.reshape(NC, 2, NPAD, 1)

    feat, ndst = pl.pallas_call(
        _scale_body,
        out_shape=(
            jax.ShapeDtypeStruct((NPAD, D), jnp.float32),
            jax.ShapeDtypeStruct((NPAD, 1), jnp.float32),
        ),
    )(histc[0, 0], histc[1, 0], histc[0, 1], histc[1, 1], h_pad)

    feat2 = jnp.broadcast_to(feat[None], (NC, NPAD, D))
    out = _agg_kernel(feat2, src2, dst2, ndst.reshape(NPAD))
    return jnp.concatenate([out[0, :HALF], out[1, HALF:N]], axis=0)


# speedup vs baseline: 2.0016x; 2.0016x over previous
import functools

import jax
import jax.numpy as jnp
from jax import lax
from jax.experimental import pallas as pl
from jax.experimental.pallas import tpu as pltpu
from jax.experimental.pallas import tpu_sc as plsc

N = 10000
D = 128
E = 320000
NC, NS = 2, 16
NW = NC * NS
EPW = E // NW
NPAD = 10240
NROWB = NPAD // 128
EPW_PAD = NPAD

CHUNK = 128
NBUF = 4
EPT = E // NS
NVEC = EPT // 16
CAP = 20480
HALF = NPAD // NC
NPASS = 2
QROWS = HALF // NPASS
ACC_ROWS = QROWS + CHUNK
RRT = QROWS // NS

_MESH = plsc.VectorSubcoreMesh(core_axis_name="c", subcore_axis_name="s")
_SC_PARAMS = pltpu.CompilerParams(
    needs_layout_passes=False, use_tc_tiling_on_sc=False
)

_GROUPS = NROWB // 8
_GBINS = NPAD // _GROUPS


def _deg_body(src_hbm, dst_hbm, out_hbm, slots_hbm, idx_v, hist, rbuf, res):
    c = lax.axis_index("c")
    s = lax.axis_index("s")
    wid = s * NC + c

    zero16 = jnp.zeros((16,), jnp.float32)
    ones = jnp.ones((16,), jnp.float32)

    @pl.loop(0, NPAD // 16)
    def _z1(i):
        hist[pl.ds(i * 16, 16)] = zero16

    pltpu.sync_copy(src_hbm.at[wid], idx_v)

    @pl.loop(0, EPW_PAD // 16)
    def _hist_src(i):
        idx = idx_v[pl.ds(i * 16, 16)]
        plsc.addupdate_scatter(hist, [idx], ones)

    pltpu.sync_copy(hist, slots_hbm.at[c, 0, s])

    @pl.loop(0, NPAD // 16)
    def _z2(i):
        hist[pl.ds(i * 16, 16)] = zero16

    pltpu.sync_copy(dst_hbm.at[wid], idx_v)

    @pl.loop(0, EPW_PAD // 16)
    def _hist_dst(i):
        idx = idx_v[pl.ds(i * 16, 16)]
        plsc.addupdate_scatter(hist, [idx], ones)

    pltpu.sync_copy(hist, slots_hbm.at[c, 1, s])

    plsc.subcore_barrier()

    @pl.when(s < _GROUPS)
    def _reduce():
        for oidx in (0, 1):
            for r in range(NS):
                pltpu.sync_copy(
                    slots_hbm.at[c, oidx, r, pl.ds(s * _GBINS, _GBINS)],
                    rbuf.at[r])

            @pl.loop(0, _GBINS // 16)
            def _sum(k):
                acc = rbuf[0, pl.ds(k * 16, 16)]
                for r in range(1, NS):
                    acc = acc + rbuf[r, pl.ds(k * 16, 16)]
                res[k // 8, pl.ds((k % 8) * 16, 16)] = acc

            pltpu.sync_copy(res, out_hbm.at[c, oidx, pl.ds(s * 8, 8)])


_deg_kernel = pl.kernel(
    _deg_body,
    out_type=(
        jax.ShapeDtypeStruct((NC, 2, NROWB, 128), jnp.float32),
        jax.ShapeDtypeStruct((NC, 2, NS, NPAD), jnp.float32),
    ),
    mesh=_MESH,
    compiler_params=_SC_PARAMS,
    scratch_types=[
        pltpu.VMEM((EPW_PAD,), jnp.int32),
        pltpu.VMEM((NPAD,), jnp.float32),
        pltpu.VMEM((NS, _GBINS), jnp.float32),
        pltpu.VMEM((8, 128), jnp.float32),
    ],
)


def _agg_body(feat_hbm, src_hbm, dst_hbm, ndst_hbm, out_hbm,
              sbuf, dbuf, didx2, nbuf, off_ref,
              rows0, rows1, rows2, rows3, acc, *sems):
    c = lax.axis_index("c")
    s = lax.axis_index("s")
    rows = (rows0, rows1, rows2, rows3)
    gsem = sems[:NBUF]
    ssem = sems[NBUF:]

    zero16 = jnp.zeros((16,), jnp.float32)
    iota = lax.iota(jnp.int32, 16)

    def stage_didx(k, b):
        for x in range(CHUNK // 16):
            didx2[b, pl.ds(x * 16, 16)] = dbuf[pl.ds(k * CHUNK + x * 16, 16)]

    def gather(k, b):
        return pltpu.async_copy(
            feat_hbm.at[c].at[sbuf.at[pl.ds(k * CHUNK, CHUNK)]],
            rows[b], gsem[b])

    def wait_gather(b):
        pltpu.make_async_copy(
            feat_hbm.at[c].at[sbuf.at[pl.ds(0, CHUNK)]],
            rows[b], gsem[b]).wait()

    def scatter(b):
        return pltpu.async_copy(
            rows[b], acc.at[didx2.at[b]], ssem[b], add=True)

    def wait_scatter(b):
        pltpu.make_async_copy(
            rows[b], acc.at[didx2.at[b]], ssem[b]).wait()

    for p in range(NPASS):
        for b in (0, 1):
            @pl.loop(0, CHUNK)
            def _zb(r):
                for x in range(D // 16):
                    rows[b][r, pl.ds(x * 16, 16)] = zero16

        zrows = ACC_ROWS // NS
        pltpu.sync_copy(rows0, acc.at[pl.ds(s * zrows, CHUNK)])
        pltpu.sync_copy(rows1.at[pl.ds(0, zrows - CHUNK)],
                        acc.at[pl.ds(s * zrows + CHUNK, zrows - CHUNK)])

        pltpu.sync_copy(src_hbm.at[s], sbuf.at[pl.ds(0, EPT)])
        pltpu.sync_copy(dst_hbm.at[s], dbuf.at[pl.ds(0, EPT)])
        base = c * HALF + p * QROWS
        pltpu.sync_copy(ndst_hbm.at[pl.ds(base + s * RRT, RRT)],
                        nbuf.at[pl.ds(0, RRT)])

        pad_s = iota + s * 16
        off_ref[0] = 0

        @pl.loop(0, NVEC)
        def _compact(i):
            off = off_ref[0]
            sv = sbuf[pl.ds(i * 16, 16)]
            dv = dbuf[pl.ds(i * 16, 16)] - base
            mi = jnp.logical_and(dv >= 0, dv < QROWS)
            inc = plsc.cumsum(mi.astype(jnp.int32))
            pos = off + inc - mi.astype(jnp.int32)
            plsc.store_scatter(sbuf, [pos], sv, mask=mi)
            plsc.store_scatter(dbuf, [pos], dv, mask=mi)
            off_ref[0] = off + inc[15]

        off = off_ref[0]
        total = jnp.maximum(
            ((off + NBUF * CHUNK - 1) // (NBUF * CHUNK)) * (NBUF * CHUNK),
            NBUF * CHUNK)
        nch = total // CHUNK

        for t in range(NBUF * CHUNK // 16):
            @pl.when(off + t * 16 < total)
            def _():
                sbuf[pl.ds(off + t * 16, 16)] = pad_s
                dbuf[pl.ds(off + t * 16, 16)] = (
                    QROWS + ((pad_s + t * 16) & (CHUNK - 1)))

        plsc.subcore_barrier()

        gather(0, 0)

        @pl.loop(0, CAP // CHUNK, step=NBUF)
        def _edges(m):
            for b in range(NBUF):
                k = m + b
                bn = (b + 1) % NBUF

                @pl.when(k < nch)
                def _():
                    wait_gather(b)
                    stage_didx(k, b)
                    scatter(b)

                    @pl.when(k + 1 - NBUF >= 0)
                    def _():
                        wait_scatter(bn)

                    @pl.when(k + 1 < nch)
                    def _():
                        gather(k + 1, bn)

        for b in range(1, NBUF):
            wait_scatter(b)

        plsc.subcore_barrier()

        for off_b, nb in ((0, CHUNK), (CHUNK, RRT - CHUNK)):
            pltpu.sync_copy(acc.at[pl.ds(s * RRT + off_b, nb)],
                            rows0.at[pl.ds(0, nb)])

            @pl.loop(0, nb)
            def _scale(r):
                nv = nbuf[pl.ds(off_b + r, 16)]
                sc = nv[0]
                for x in range(D // 16):
                    rows0[r, pl.ds(x * 16, 16)] = (
                        rows0[r, pl.ds(x * 16, 16)] * sc)

            pltpu.sync_copy(
                rows0.at[pl.ds(0, nb)],
                out_hbm.at[c, pl.ds(base + s * RRT + off_b, nb)])


_agg_kernel = pl.kernel(
    _agg_body,
    out_type=jax.ShapeDtypeStruct((NC, NPAD, D), jnp.float32),
    mesh=_MESH,
    compiler_params=_SC_PARAMS,
    scratch_types=[
        pltpu.VMEM((CAP,), jnp.int32),
        pltpu.VMEM((CAP,), jnp.int32),
        pltpu.VMEM((NBUF, CHUNK), jnp.int32),
        pltpu.VMEM((RRT + 16,), jnp.float32),
        pltpu.SMEM((1,), jnp.int32),
        pltpu.VMEM((CHUNK, D), jnp.float32),
        pltpu.VMEM((CHUNK, D), jnp.float32),
        pltpu.VMEM((CHUNK, D), jnp.float32),
        pltpu.VMEM((CHUNK, D), jnp.float32),
        pltpu.VMEM_SHARED((ACC_ROWS, D), jnp.float32),
    ]
    + [pltpu.SemaphoreType.DMA] * (2 * NBUF),
)


def _scale_body(hs0, hs1, hd0, hd1, h_ref, feat_ref, ndst_ref):
    out_deg = hs0[...] + hs1[...]
    norm_src = 1.0 / jnp.maximum(out_deg, 1.0)
    feat_ref[...] = h_ref[...] * norm_src
    in_deg = hd0[...] + hd1[...]
    ndst_ref[...] = 1.0 / jnp.maximum(in_deg, 1.0)


def kernel(h, edge_index):
    ei = edge_index.astype(jnp.int32)
    src = ei[0]
    dst = ei[1]
    pad_idx = N + (jnp.arange(EPW_PAD - EPW, dtype=jnp.int32) % (NPAD - N))
    pad_blk = jnp.broadcast_to(pad_idx, (NW, EPW_PAD - EPW))
    src_p = jnp.concatenate([src.reshape(NW, EPW), pad_blk], axis=1)
    dst_p = jnp.concatenate([dst.reshape(NW, EPW), pad_blk], axis=1)
    src2 = src.reshape(NS, EPT)
    dst2 = dst.reshape(NS, EPT)
    h_pad = jnp.pad(h, ((0, NPAD - N), (0, 0)))

    hist, _ = _deg_kernel(src_p, dst_p)
    histc = hist.reshape(NC, 2, NPAD, 1)

    feat, ndst = pl.pallas_call(
        _scale_body,
        out_shape=(
            jax.ShapeDtypeStruct((NPAD, D), jnp.float32),
            jax.ShapeDtypeStruct((NPAD, 1), jnp.float32),
        ),
    )(histc[0, 0], histc[1, 0], histc[0, 1], histc[1, 1], h_pad)

    feat2 = jnp.broadcast_to(feat[None], (NC, NPAD, D))
    out = _agg_kernel(feat2, src2, dst2, ndst.reshape(NPAD))
    return jnp.concatenate([out[0, :HALF], out[1, HALF:N]], axis=0)


# speedup vs baseline: 2.0920x; 1.0451x over previous
import functools

import jax
import jax.numpy as jnp
from jax import lax
from jax.experimental import pallas as pl
from jax.experimental.pallas import tpu as pltpu
from jax.experimental.pallas import tpu_sc as plsc

N = 10000
D = 128
E = 320000
NC, NS = 2, 16
NW = NC * NS
EPW = E // NW
NPAD = 10240
NROWB = NPAD // 128
EPW_PAD = NPAD

CHUNK = 128
NBUF = 4
EPT = E // NS
NVEC = EPT // 16
CAP = 20480
HALF = NPAD // NC
NPASS = 2
QROWS = HALF // NPASS
ACC_ROWS = QROWS + CHUNK
RRT = QROWS // NS

_MESH = plsc.VectorSubcoreMesh(core_axis_name="c", subcore_axis_name="s")
_SC_PARAMS = pltpu.CompilerParams(
    needs_layout_passes=False, use_tc_tiling_on_sc=False
)

_GROUPS = NROWB // 8
_GBINS = NPAD // _GROUPS


def _deg_body(src_hbm, dst_hbm, out_hbm, idx_v, hist, rbuf, res,
              hs_sh, hd_sh):
    c = lax.axis_index("c")
    s = lax.axis_index("s")
    wid = s * NC + c

    zero16 = jnp.zeros((16,), jnp.float32)
    ones = jnp.ones((16,), jnp.float32)

    @pl.loop(0, NPAD // 16)
    def _z1(i):
        hist[pl.ds(i * 16, 16)] = zero16

    pltpu.sync_copy(src_hbm.at[wid], idx_v)

    @pl.loop(0, EPW_PAD // 16)
    def _hist_src(i):
        idx = idx_v[pl.ds(i * 16, 16)]
        plsc.addupdate_scatter(hist, [idx], ones)

    pltpu.sync_copy(hist, hs_sh.at[s])

    @pl.loop(0, NPAD // 16)
    def _z2(i):
        hist[pl.ds(i * 16, 16)] = zero16

    pltpu.sync_copy(dst_hbm.at[wid], idx_v)

    @pl.loop(0, EPW_PAD // 16)
    def _hist_dst(i):
        idx = idx_v[pl.ds(i * 16, 16)]
        plsc.addupdate_scatter(hist, [idx], ones)

    pltpu.sync_copy(hist, hd_sh.at[s])

    plsc.subcore_barrier()

    @pl.when(s < _GROUPS)
    def _reduce():
        for oidx, sh in ((0, hs_sh), (1, hd_sh)):
            for r in range(NS):
                pltpu.sync_copy(sh.at[r, pl.ds(s * _GBINS, _GBINS)],
                                rbuf.at[r])

            @pl.loop(0, _GBINS // 16)
            def _sum(k):
                acc = rbuf[0, pl.ds(k * 16, 16)]
                for r in range(1, NS):
                    acc = acc + rbuf[r, pl.ds(k * 16, 16)]
                res[k // 8, pl.ds((k % 8) * 16, 16)] = acc

            pltpu.sync_copy(res, out_hbm.at[c, oidx, pl.ds(s * 8, 8)])


_deg_kernel = pl.kernel(
    _deg_body,
    out_type=jax.ShapeDtypeStruct((NC, 2, NROWB, 128), jnp.float32),
    mesh=_MESH,
    compiler_params=_SC_PARAMS,
    scratch_types=[
        pltpu.VMEM((EPW_PAD,), jnp.int32),
        pltpu.VMEM((NPAD,), jnp.float32),
        pltpu.VMEM((NS, _GBINS), jnp.float32),
        pltpu.VMEM((8, 128), jnp.float32),
        pltpu.VMEM_SHARED((NS, NPAD), jnp.float32),
        pltpu.VMEM_SHARED((NS, NPAD), jnp.float32),
    ],
)


def _agg_body(feat_hbm, src_hbm, dst_hbm, ndst_hbm, out_hbm,
              sbuf, dbuf, didx2, nbuf, off_ref,
              rows0, rows1, rows2, rows3, acc, *sems):
    c = lax.axis_index("c")
    s = lax.axis_index("s")
    rows = (rows0, rows1, rows2, rows3)
    gsem = sems[:NBUF]
    ssem = sems[NBUF:]

    zero16 = jnp.zeros((16,), jnp.float32)
    iota = lax.iota(jnp.int32, 16)

    def stage_didx(k, b):
        for x in range(CHUNK // 16):
            didx2[b, pl.ds(x * 16, 16)] = dbuf[pl.ds(k * CHUNK + x * 16, 16)]

    def gather(k, b):
        return pltpu.async_copy(
            feat_hbm.at[c].at[sbuf.at[pl.ds(k * CHUNK, CHUNK)]],
            rows[b], gsem[b])

    def wait_gather(b):
        pltpu.make_async_copy(
            feat_hbm.at[c].at[sbuf.at[pl.ds(0, CHUNK)]],
            rows[b], gsem[b]).wait()

    def scatter(b):
        return pltpu.async_copy(
            rows[b], acc.at[didx2.at[b]], ssem[b], add=True)

    def wait_scatter(b):
        pltpu.make_async_copy(
            rows[b], acc.at[didx2.at[b]], ssem[b]).wait()

    for p in range(NPASS):
        for b in (0, 1):
            @pl.loop(0, CHUNK)
            def _zb(r):
                for x in range(D // 16):
                    rows[b][r, pl.ds(x * 16, 16)] = zero16

        zrows = ACC_ROWS // NS
        pltpu.sync_copy(rows0, acc.at[pl.ds(s * zrows, CHUNK)])
        pltpu.sync_copy(rows1.at[pl.ds(0, zrows - CHUNK)],
                        acc.at[pl.ds(s * zrows + CHUNK, zrows - CHUNK)])

        pltpu.sync_copy(src_hbm.at[s], sbuf.at[pl.ds(0, EPT)])
        pltpu.sync_copy(dst_hbm.at[s], dbuf.at[pl.ds(0, EPT)])
        base = c * HALF + p * QROWS
        pltpu.sync_copy(ndst_hbm.at[pl.ds(base + s * RRT, RRT)],
                        nbuf.at[pl.ds(0, RRT)])

        pad_s = iota + s * 16
        off_ref[0] = 0

        @pl.loop(0, NVEC)
        def _compact(i):
            off = off_ref[0]
            sv = sbuf[pl.ds(i * 16, 16)]
            dv = dbuf[pl.ds(i * 16, 16)] - base
            mi = jnp.logical_and(dv >= 0, dv < QROWS)
            inc = plsc.cumsum(mi.astype(jnp.int32))
            pos = off + inc - mi.astype(jnp.int32)
            plsc.store_scatter(sbuf, [pos], sv, mask=mi)
            plsc.store_scatter(dbuf, [pos], dv, mask=mi)
            off_ref[0] = off + inc[15]

        off = off_ref[0]
        total = jnp.maximum(
            ((off + NBUF * CHUNK - 1) // (NBUF * CHUNK)) * (NBUF * CHUNK),
            NBUF * CHUNK)
        nch = total // CHUNK

        for t in range(NBUF * CHUNK // 16):
            @pl.when(off + t * 16 < total)
            def _():
                sbuf[pl.ds(off + t * 16, 16)] = pad_s
                dbuf[pl.ds(off + t * 16, 16)] = (
                    QROWS + ((pad_s + t * 16) & (CHUNK - 1)))

        plsc.subcore_barrier()

        gather(0, 0)

        @pl.loop(0, CAP // CHUNK, step=NBUF)
        def _edges(m):
            for b in range(NBUF):
                k = m + b
                bn = (b + 1) % NBUF

                @pl.when(k < nch)
                def _():
                    wait_gather(b)
                    stage_didx(k, b)
                    scatter(b)

                    @pl.when(k + 1 - NBUF >= 0)
                    def _():
                        wait_scatter(bn)

                    @pl.when(k + 1 < nch)
                    def _():
                        gather(k + 1, bn)

        for b in range(1, NBUF):
            wait_scatter(b)

        plsc.subcore_barrier()

        for off_b, nb in ((0, CHUNK), (CHUNK, RRT - CHUNK)):
            pltpu.sync_copy(acc.at[pl.ds(s * RRT + off_b, nb)],
                            rows0.at[pl.ds(0, nb)])

            @pl.loop(0, nb)
            def _scale(r):
                nv = nbuf[pl.ds(off_b + r, 16)]
                sc = nv[0]
                for x in range(D // 16):
                    rows0[r, pl.ds(x * 16, 16)] = (
                        rows0[r, pl.ds(x * 16, 16)] * sc)

            pltpu.sync_copy(
                rows0.at[pl.ds(0, nb)],
                out_hbm.at[c, pl.ds(base + s * RRT + off_b, nb)])


_agg_kernel = pl.kernel(
    _agg_body,
    out_type=jax.ShapeDtypeStruct((NC, NPAD, D), jnp.float32),
    mesh=_MESH,
    compiler_params=_SC_PARAMS,
    scratch_types=[
        pltpu.VMEM((CAP,), jnp.int32),
        pltpu.VMEM((CAP,), jnp.int32),
        pltpu.VMEM((NBUF, CHUNK), jnp.int32),
        pltpu.VMEM((RRT + 16,), jnp.float32),
        pltpu.SMEM((1,), jnp.int32),
        pltpu.VMEM((CHUNK, D), jnp.float32),
        pltpu.VMEM((CHUNK, D), jnp.float32),
        pltpu.VMEM((CHUNK, D), jnp.float32),
        pltpu.VMEM((CHUNK, D), jnp.float32),
        pltpu.VMEM_SHARED((ACC_ROWS, D), jnp.float32),
    ]
    + [pltpu.SemaphoreType.DMA] * (2 * NBUF),
)


def _scale_body(hs0, hs1, hd0, hd1, h_ref, feat_ref, ndst_ref):
    out_deg = hs0[...] + hs1[...]
    norm_src = 1.0 / jnp.maximum(out_deg, 1.0)
    feat_ref[...] = h_ref[...] * norm_src
    in_deg = hd0[...] + hd1[...]
    ndst_ref[...] = 1.0 / jnp.maximum(in_deg, 1.0)


def kernel(h, edge_index):
    ei = edge_index.astype(jnp.int32)
    src = ei[0]
    dst = ei[1]
    pad_idx = N + (jnp.arange(EPW_PAD - EPW, dtype=jnp.int32) % (NPAD - N))
    pad_blk = jnp.broadcast_to(pad_idx, (NW, EPW_PAD - EPW))
    src_p = jnp.concatenate([src.reshape(NW, EPW), pad_blk], axis=1)
    dst_p = jnp.concatenate([dst.reshape(NW, EPW), pad_blk], axis=1)
    src2 = src.reshape(NS, EPT)
    dst2 = dst.reshape(NS, EPT)
    h_pad = jnp.pad(h, ((0, NPAD - N), (0, 0)))

    hist = _deg_kernel(src_p, dst_p)
    histc = hist.reshape(NC, 2, NPAD, 1)

    feat, ndst = pl.pallas_call(
        _scale_body,
        out_shape=(
            jax.ShapeDtypeStruct((NPAD, D), jnp.float32),
            jax.ShapeDtypeStruct((NPAD, 1), jnp.float32),
        ),
    )(histc[0, 0], histc[1, 0], histc[0, 1], histc[1, 1], h_pad)

    feat2 = jnp.broadcast_to(feat[None], (NC, NPAD, D))
    out = _agg_kernel(feat2, src2, dst2, ndst.reshape(NPAD))
    return jnp.concatenate([out[0, :HALF], out[1, HALF:N]], axis=0)


# speedup vs baseline: 2.1227x; 1.0147x over previous
import functools

import jax
import jax.numpy as jnp
from jax import lax
from jax.experimental import pallas as pl
from jax.experimental.pallas import tpu as pltpu
from jax.experimental.pallas import tpu_sc as plsc

N = 10000
D = 128
E = 320000
NC, NS = 2, 16
NW = NC * NS
EPW = E // NW
NPAD = 10240
NROWB = NPAD // 128

CHUNK = 128
NBUF = 4
EPT = E // NS
NVEC = EPT // 16
CAP = 20480
HALF = NPAD // NC
NPASS = 2
QROWS = HALF // NPASS
ACC_ROWS = QROWS + CHUNK
RRT = QROWS // NS

_MESH = plsc.VectorSubcoreMesh(core_axis_name="c", subcore_axis_name="s")
_SC_PARAMS = pltpu.CompilerParams(
    needs_layout_passes=False, use_tc_tiling_on_sc=False
)

_GROUPS = NROWB // 8
_GBINS = NPAD // _GROUPS


def _deg_body(src_hbm, dst_hbm, out_hbm, idx_v, hist, rbuf, res,
              hs_sh, hd_sh):
    c = lax.axis_index("c")
    s = lax.axis_index("s")
    wid = s * NC + c

    zero16 = jnp.zeros((16,), jnp.float32)
    ones = jnp.ones((16,), jnp.float32)

    @pl.loop(0, NPAD // 16)
    def _z1(i):
        hist[pl.ds(i * 16, 16)] = zero16

    pltpu.sync_copy(src_hbm.at[pl.ds(wid * EPW, EPW)], idx_v)

    @pl.loop(0, EPW // 16)
    def _hist_src(i):
        idx = idx_v[pl.ds(i * 16, 16)]
        plsc.addupdate_scatter(hist, [idx], ones)

    pltpu.sync_copy(hist, hs_sh.at[s])

    @pl.loop(0, NPAD // 16)
    def _z2(i):
        hist[pl.ds(i * 16, 16)] = zero16

    pltpu.sync_copy(dst_hbm.at[pl.ds(wid * EPW, EPW)], idx_v)

    @pl.loop(0, EPW // 16)
    def _hist_dst(i):
        idx = idx_v[pl.ds(i * 16, 16)]
        plsc.addupdate_scatter(hist, [idx], ones)

    pltpu.sync_copy(hist, hd_sh.at[s])

    plsc.subcore_barrier()

    @pl.when(s < _GROUPS)
    def _reduce():
        for oidx, sh in ((0, hs_sh), (1, hd_sh)):
            for r in range(NS):
                pltpu.sync_copy(sh.at[r, pl.ds(s * _GBINS, _GBINS)],
                                rbuf.at[r])

            @pl.loop(0, _GBINS // 16)
            def _sum(k):
                acc = rbuf[0, pl.ds(k * 16, 16)]
                for r in range(1, NS):
                    acc = acc + rbuf[r, pl.ds(k * 16, 16)]
                res[k // 8, pl.ds((k % 8) * 16, 16)] = acc

            pltpu.sync_copy(res, out_hbm.at[c, oidx, pl.ds(s * 8, 8)])


_deg_kernel = pl.kernel(
    _deg_body,
    out_type=jax.ShapeDtypeStruct((NC, 2, NROWB, 128), jnp.float32),
    mesh=_MESH,
    compiler_params=_SC_PARAMS,
    scratch_types=[
        pltpu.VMEM((EPW,), jnp.int32),
        pltpu.VMEM((NPAD,), jnp.float32),
        pltpu.VMEM((NS, _GBINS), jnp.float32),
        pltpu.VMEM((8, 128), jnp.float32),
        pltpu.VMEM_SHARED((NS, NPAD), jnp.float32),
        pltpu.VMEM_SHARED((NS, NPAD), jnp.float32),
    ],
)


def _agg_body(feat_hbm, src_hbm, dst_hbm, ndst_hbm, out_hbm,
              sbuf, dbuf, didx2, nbuf, off_ref,
              rows0, rows1, rows2, rows3, acc, *sems):
    c = lax.axis_index("c")
    s = lax.axis_index("s")
    rows = (rows0, rows1, rows2, rows3)
    gsem = sems[:NBUF]
    ssem = sems[NBUF:]

    zero16 = jnp.zeros((16,), jnp.float32)
    iota = lax.iota(jnp.int32, 16)

    def stage_didx(k, b):
        for x in range(CHUNK // 16):
            didx2[b, pl.ds(x * 16, 16)] = dbuf[pl.ds(k * CHUNK + x * 16, 16)]

    def gather(k, b):
        return pltpu.async_copy(
            feat_hbm.at[c].at[sbuf.at[pl.ds(k * CHUNK, CHUNK)]],
            rows[b], gsem[b])

    def wait_gather(b):
        pltpu.make_async_copy(
            feat_hbm.at[c].at[sbuf.at[pl.ds(0, CHUNK)]],
            rows[b], gsem[b]).wait()

    def scatter(b):
        return pltpu.async_copy(
            rows[b], acc.at[didx2.at[b]], ssem[b], add=True)

    def wait_scatter(b):
        pltpu.make_async_copy(
            rows[b], acc.at[didx2.at[b]], ssem[b]).wait()

    for p in range(NPASS):
        for b in (0, 1):
            @pl.loop(0, CHUNK)
            def _zb(r):
                for x in range(D // 16):
                    rows[b][r, pl.ds(x * 16, 16)] = zero16

        zrows = ACC_ROWS // NS
        pltpu.sync_copy(rows0, acc.at[pl.ds(s * zrows, CHUNK)])
        pltpu.sync_copy(rows1.at[pl.ds(0, zrows - CHUNK)],
                        acc.at[pl.ds(s * zrows + CHUNK, zrows - CHUNK)])

        pltpu.sync_copy(src_hbm.at[s], sbuf.at[pl.ds(0, EPT)])
        pltpu.sync_copy(dst_hbm.at[s], dbuf.at[pl.ds(0, EPT)])
        base = c * HALF + p * QROWS
        pltpu.sync_copy(ndst_hbm.at[pl.ds(base + s * RRT, RRT)],
                        nbuf.at[pl.ds(0, RRT)])

        pad_s = iota + s * 16
        off_ref[0] = 0

        @pl.loop(0, NVEC)
        def _compact(i):
            off = off_ref[0]
            sv = sbuf[pl.ds(i * 16, 16)]
            dv = dbuf[pl.ds(i * 16, 16)] - base
            mi = jnp.logical_and(dv >= 0, dv < QROWS)
            inc = plsc.cumsum(mi.astype(jnp.int32))
            pos = off + inc - mi.astype(jnp.int32)
            plsc.store_scatter(sbuf, [pos], sv, mask=mi)
            plsc.store_scatter(dbuf, [pos], dv, mask=mi)
            off_ref[0] = off + inc[15]

        off = off_ref[0]
        total = jnp.maximum(
            ((off + NBUF * CHUNK - 1) // (NBUF * CHUNK)) * (NBUF * CHUNK),
            NBUF * CHUNK)
        nch = total // CHUNK

        for t in range(NBUF * CHUNK // 16):
            @pl.when(off + t * 16 < total)
            def _():
                sbuf[pl.ds(off + t * 16, 16)] = pad_s
                dbuf[pl.ds(off + t * 16, 16)] = (
                    QROWS + ((pad_s + t * 16) & (CHUNK - 1)))

        plsc.subcore_barrier()

        gather(0, 0)

        @pl.loop(0, CAP // CHUNK, step=NBUF)
        def _edges(m):
            for b in range(NBUF):
                k = m + b
                bn = (b + 1) % NBUF

                @pl.when(k < nch)
                def _():
                    wait_gather(b)
                    stage_didx(k, b)
                    scatter(b)

                    @pl.when(k + 1 - NBUF >= 0)
                    def _():
                        wait_scatter(bn)

                    @pl.when(k + 1 < nch)
                    def _():
                        gather(k + 1, bn)

        for b in range(1, NBUF):
            wait_scatter(b)

        plsc.subcore_barrier()

        for off_b, nb in ((0, CHUNK), (CHUNK, RRT - CHUNK)):
            pltpu.sync_copy(acc.at[pl.ds(s * RRT + off_b, nb)],
                            rows0.at[pl.ds(0, nb)])

            @pl.loop(0, nb)
            def _scale(r):
                nv = nbuf[pl.ds(off_b + r, 16)]
                sc = nv[0]
                for x in range(D // 16):
                    rows0[r, pl.ds(x * 16, 16)] = (
                        rows0[r, pl.ds(x * 16, 16)] * sc)

            pltpu.sync_copy(
                rows0.at[pl.ds(0, nb)],
                out_hbm.at[c, pl.ds(base + s * RRT + off_b, nb)])


_agg_kernel = pl.kernel(
    _agg_body,
    out_type=jax.ShapeDtypeStruct((NC, NPAD, D), jnp.float32),
    mesh=_MESH,
    compiler_params=_SC_PARAMS,
    scratch_types=[
        pltpu.VMEM((CAP,), jnp.int32),
        pltpu.VMEM((CAP,), jnp.int32),
        pltpu.VMEM((NBUF, CHUNK), jnp.int32),
        pltpu.VMEM((RRT + 16,), jnp.float32),
        pltpu.SMEM((1,), jnp.int32),
        pltpu.VMEM((CHUNK, D), jnp.float32),
        pltpu.VMEM((CHUNK, D), jnp.float32),
        pltpu.VMEM((CHUNK, D), jnp.float32),
        pltpu.VMEM((CHUNK, D), jnp.float32),
        pltpu.VMEM_SHARED((ACC_ROWS, D), jnp.float32),
    ]
    + [pltpu.SemaphoreType.DMA] * (2 * NBUF),
)


def _scale_body(hs0, hs1, hd0, hd1, h_ref, feat_ref, ndst_ref):
    out_deg = hs0[...] + hs1[...]
    norm_src = 1.0 / jnp.maximum(out_deg, 1.0)
    feat_ref[...] = h_ref[...] * norm_src
    in_deg = hd0[...] + hd1[...]
    ndst_ref[...] = 1.0 / jnp.maximum(in_deg, 1.0)


def kernel(h, edge_index):
    ei = edge_index.astype(jnp.int32)
    src = ei[0]
    dst = ei[1]
    src2 = src.reshape(NS, EPT)
    dst2 = dst.reshape(NS, EPT)
    h_pad = jnp.pad(h, ((0, NPAD - N), (0, 0)))

    hist = _deg_kernel(src, dst)
    histc = hist.reshape(NC, 2, NPAD, 1)

    feat, ndst = pl.pallas_call(
        _scale_body,
        out_shape=(
            jax.ShapeDtypeStruct((NPAD, D), jnp.float32),
            jax.ShapeDtypeStruct((NPAD, 1), jnp.float32),
        ),
    )(histc[0, 0], histc[1, 0], histc[0, 1], histc[1, 1], h_pad)

    feat2 = jnp.broadcast_to(feat[None], (NC, NPAD, D))
    out = _agg_kernel(feat2, src2, dst2, ndst.reshape(NPAD))
    return jnp.concatenate([out[0, :HALF], out[1, HALF:N]], axis=0)


# speedup vs baseline: 2.1380x; 1.0072x over previous
import functools

import jax
import jax.numpy as jnp
from jax import lax
from jax.experimental import pallas as pl
from jax.experimental.pallas import tpu as pltpu
from jax.experimental.pallas import tpu_sc as plsc

N = 10000
D = 128
E = 320000
NC, NS = 2, 16
NW = NC * NS
EPW = E // NW
NPAD = 10240
NROWB = NPAD // 128

CHUNK = 128
NBUF = 4
EPT = E // NS
NVEC = EPT // 16
CAP = 20480
HALF = NPAD // NC
NPASS = 2
QROWS = HALF // NPASS
ACC_ROWS = QROWS + CHUNK
RRT = QROWS // NS

_MESH = plsc.VectorSubcoreMesh(core_axis_name="c", subcore_axis_name="s")
_SC_PARAMS = pltpu.CompilerParams(
    needs_layout_passes=False, use_tc_tiling_on_sc=False
)

_GROUPS = NROWB // 8
_GBINS = NPAD // _GROUPS


def _deg_body(src_hbm, dst_hbm, out_hbm, idx_v, hist, rbuf, res,
              hs_sh, hd_sh):
    c = lax.axis_index("c")
    s = lax.axis_index("s")
    wid = s * NC + c

    zero16 = jnp.zeros((16,), jnp.float32)
    ones = jnp.ones((16,), jnp.float32)

    @pl.loop(0, NPAD // 16)
    def _z1(i):
        hist[pl.ds(i * 16, 16)] = zero16

    pltpu.sync_copy(src_hbm.at[pl.ds(wid * EPW, EPW)], idx_v)

    @pl.loop(0, EPW // 16)
    def _hist_src(i):
        idx = idx_v[pl.ds(i * 16, 16)]
        plsc.addupdate_scatter(hist, [idx], ones)

    pltpu.sync_copy(hist, hs_sh.at[s])

    @pl.loop(0, NPAD // 16)
    def _z2(i):
        hist[pl.ds(i * 16, 16)] = zero16

    pltpu.sync_copy(dst_hbm.at[pl.ds(wid * EPW, EPW)], idx_v)

    @pl.loop(0, EPW // 16)
    def _hist_dst(i):
        idx = idx_v[pl.ds(i * 16, 16)]
        plsc.addupdate_scatter(hist, [idx], ones)

    pltpu.sync_copy(hist, hd_sh.at[s])

    plsc.subcore_barrier()

    @pl.when(s < _GROUPS)
    def _reduce():
        for oidx, sh in ((0, hs_sh), (1, hd_sh)):
            for r in range(NS):
                pltpu.sync_copy(sh.at[r, pl.ds(s * _GBINS, _GBINS)],
                                rbuf.at[r])

            @pl.loop(0, _GBINS // 16)
            def _sum(k):
                acc = rbuf[0, pl.ds(k * 16, 16)]
                for r in range(1, NS):
                    acc = acc + rbuf[r, pl.ds(k * 16, 16)]
                res[k // 8, pl.ds((k % 8) * 16, 16)] = acc

            pltpu.sync_copy(res, out_hbm.at[c, oidx, pl.ds(s * 8, 8)])


_deg_kernel = pl.kernel(
    _deg_body,
    out_type=jax.ShapeDtypeStruct((NC, 2, NROWB, 128), jnp.float32),
    mesh=_MESH,
    compiler_params=_SC_PARAMS,
    scratch_types=[
        pltpu.VMEM((EPW,), jnp.int32),
        pltpu.VMEM((NPAD,), jnp.float32),
        pltpu.VMEM((NS, _GBINS), jnp.float32),
        pltpu.VMEM((8, 128), jnp.float32),
        pltpu.VMEM_SHARED((NS, NPAD), jnp.float32),
        pltpu.VMEM_SHARED((NS, NPAD), jnp.float32),
    ],
)


def _agg_body(feat_hbm, src_hbm, dst_hbm, ndst_hbm, out_hbm,
              sbuf, dbuf, didx2, nbuf, off_ref,
              rows0, rows1, rows2, rows3, acc, *sems):
    c = lax.axis_index("c")
    s = lax.axis_index("s")
    rows = (rows0, rows1, rows2, rows3)
    gsem = sems[:NBUF]
    ssem = sems[NBUF:]

    zero16 = jnp.zeros((16,), jnp.float32)
    iota = lax.iota(jnp.int32, 16)

    def stage_didx(k, b):
        for x in range(CHUNK // 16):
            didx2[b, pl.ds(x * 16, 16)] = dbuf[pl.ds(k * CHUNK + x * 16, 16)]

    def gather(k, b):
        return pltpu.async_copy(
            feat_hbm.at[c].at[sbuf.at[pl.ds(k * CHUNK, CHUNK)]],
            rows[b], gsem[b])

    def wait_gather(b):
        pltpu.make_async_copy(
            feat_hbm.at[c].at[sbuf.at[pl.ds(0, CHUNK)]],
            rows[b], gsem[b]).wait()

    def scatter(b):
        return pltpu.async_copy(
            rows[b], acc.at[didx2.at[b]], ssem[b], add=True)

    def wait_scatter(b):
        pltpu.make_async_copy(
            rows[b], acc.at[didx2.at[b]], ssem[b]).wait()

    for p in range(NPASS):
        for b in (0, 1):
            @pl.loop(0, CHUNK)
            def _zb(r):
                for x in range(D // 16):
                    rows[b][r, pl.ds(x * 16, 16)] = zero16

        zrows = ACC_ROWS // NS
        pltpu.sync_copy(rows0, acc.at[pl.ds(s * zrows, CHUNK)])
        pltpu.sync_copy(rows1.at[pl.ds(0, zrows - CHUNK)],
                        acc.at[pl.ds(s * zrows + CHUNK, zrows - CHUNK)])

        pltpu.sync_copy(src_hbm.at[s], sbuf.at[pl.ds(0, EPT)])
        pltpu.sync_copy(dst_hbm.at[s], dbuf.at[pl.ds(0, EPT)])
        base = c * HALF + p * QROWS
        pltpu.sync_copy(ndst_hbm.at[pl.ds(base + s * RRT, RRT)],
                        nbuf.at[pl.ds(0, RRT)])

        pad_s = iota + s * 16
        off_ref[0] = 0

        @pl.loop(0, NVEC)
        def _compact(i):
            off = off_ref[0]
            sv = sbuf[pl.ds(i * 16, 16)]
            dv = dbuf[pl.ds(i * 16, 16)] - base
            mi = jnp.logical_and(dv >= 0, dv < QROWS)
            inc = plsc.cumsum(mi.astype(jnp.int32))
            pos = off + inc - mi.astype(jnp.int32)
            plsc.store_scatter(sbuf, [pos], sv, mask=mi)
            plsc.store_scatter(dbuf, [pos], dv, mask=mi)
            off_ref[0] = off + inc[15]

        off = off_ref[0]
        total = jnp.maximum(
            ((off + NBUF * CHUNK - 1) // (NBUF * CHUNK)) * (NBUF * CHUNK),
            NBUF * CHUNK)
        nch = total // CHUNK

        for t in range(NBUF * CHUNK // 16):
            @pl.when(off + t * 16 < total)
            def _():
                sbuf[pl.ds(off + t * 16, 16)] = pad_s
                dbuf[pl.ds(off + t * 16, 16)] = (
                    QROWS + ((pad_s + t * 16) & (CHUNK - 1)))

        plsc.subcore_barrier()

        gather(0, 0)

        @pl.loop(0, CAP // CHUNK, step=NBUF)
        def _edges(m):
            for b in range(NBUF):
                k = m + b
                bn = (b + 1) % NBUF

                @pl.when(k < nch)
                def _():
                    wait_gather(b)
                    stage_didx(k, b)
                    scatter(b)

                    @pl.when(k + 1 - NBUF >= 0)
                    def _():
                        wait_scatter(bn)

                    @pl.when(k + 1 < nch)
                    def _():
                        gather(k + 1, bn)

        for b in range(1, NBUF):
            wait_scatter(b)

        plsc.subcore_barrier()

        for off_b, nb in ((0, CHUNK), (CHUNK, RRT - CHUNK)):
            pltpu.sync_copy(acc.at[pl.ds(s * RRT + off_b, nb)],
                            rows0.at[pl.ds(0, nb)])

            @pl.loop(0, nb)
            def _scale(r):
                nv = nbuf[pl.ds(off_b + r, 16)]
                sc = nv[0]
                for x in range(D // 16):
                    rows0[r, pl.ds(x * 16, 16)] = (
                        rows0[r, pl.ds(x * 16, 16)] * sc)

            pltpu.sync_copy(
                rows0.at[pl.ds(0, nb)],
                out_hbm.at[c, pl.ds(base + s * RRT + off_b, nb)])


_agg_kernel = pl.kernel(
    _agg_body,
    out_type=jax.ShapeDtypeStruct((NC, NPAD, D), jnp.float32),
    mesh=_MESH,
    compiler_params=_SC_PARAMS,
    scratch_types=[
        pltpu.VMEM((CAP,), jnp.int32),
        pltpu.VMEM((CAP,), jnp.int32),
        pltpu.VMEM((NBUF, CHUNK), jnp.int32),
        pltpu.VMEM((RRT + 16,), jnp.float32),
        pltpu.SMEM((1,), jnp.int32),
        pltpu.VMEM((CHUNK, D), jnp.float32),
        pltpu.VMEM((CHUNK, D), jnp.float32),
        pltpu.VMEM((CHUNK, D), jnp.float32),
        pltpu.VMEM((CHUNK, D), jnp.float32),
        pltpu.VMEM_SHARED((ACC_ROWS, D), jnp.float32),
    ]
    + [pltpu.SemaphoreType.DMA] * (2 * NBUF),
)


def _scale_body(hs0, hs1, hd0, hd1, h_ref, feat_ref, ndst_ref):
    out_deg = hs0[...] + hs1[...]
    norm_src = 1.0 / jnp.maximum(out_deg, 1.0)
    scaled = h_ref[...] * norm_src[:N]
    for cc in range(NC):
        feat_ref[cc, :N] = scaled
    in_deg = hd0[...] + hd1[...]
    ndst_ref[...] = 1.0 / jnp.maximum(in_deg, 1.0)


def kernel(h, edge_index):
    ei = edge_index.astype(jnp.int32)
    src = ei[0]
    dst = ei[1]
    src2 = src.reshape(NS, EPT)
    dst2 = dst.reshape(NS, EPT)

    hist = _deg_kernel(src, dst)
    histc = hist.reshape(NC, 2, NPAD, 1)

    feat2, ndst = pl.pallas_call(
        _scale_body,
        out_shape=(
            jax.ShapeDtypeStruct((NC, NPAD, D), jnp.float32),
            jax.ShapeDtypeStruct((NPAD, 1), jnp.float32),
        ),
    )(histc[0, 0], histc[1, 0], histc[0, 1], histc[1, 1], h)

    out = _agg_kernel(feat2, src2, dst2, ndst.reshape(NPAD))
    return jnp.concatenate([out[0, :HALF], out[1, HALF:N]], axis=0)
